# baseline (speedup 1.0000x reference)
import functools

import jax
import jax.numpy as jnp
from jax import lax
from jax.experimental import pallas as pl
from jax.experimental.pallas import tpu as pltpu
from jax.experimental.pallas import tpu_sc as plsc

_MASK_RATIO = 0.75
_LANES = 16
_FULL_STEPS = 13
_TOTAL_STEPS = 30
_SPLITS = 8


@functools.lru_cache(maxsize=None)
def _build_mask_kernel(B, N, K):
    NW = 32
    rows_per_w = B // NW
    n_chunks = N // _LANES
    q_chunks = n_chunks // _SPLITS
    q_elems = N // _SPLITS
    mesh = plsc.VectorSubcoreMesh(core_axis_name="c", subcore_axis_name="s")

    sentinel = 1 << _TOTAL_STEPS
    cand_types = [pltpu.VMEM((q_elems + _LANES,), jnp.int32)
                  for _ in range(rows_per_w * _SPLITS)]
    comb_types = [pltpu.VMEM((2 * _LANES,), jnp.int32)
                  for _ in range(rows_per_w)]

    @functools.partial(
        pl.kernel,
        mesh=mesh,
        out_type=jax.ShapeDtypeStruct((B, N), jnp.float32),
        compiler_params=pltpu.CompilerParams(needs_layout_passes=False),
        scratch_types=[
            pltpu.VMEM((rows_per_w, N), jnp.float32),
            pltpu.VMEM((rows_per_w, N), jnp.float32),
        ] + cand_types + comb_types,
    )
    def body(noise_hbm, out_hbm, noise_v, out_v, *scratch):
        cand = [scratch[r * _SPLITS:(r + 1) * _SPLITS]
                for r in range(rows_per_w)]
        comb = scratch[rows_per_w * _SPLITS:]
        wid = lax.axis_index("s") * 2 + lax.axis_index("c")
        base = wid * rows_per_w
        pltpu.sync_copy(noise_hbm.at[pl.ds(base, rows_per_w)], noise_v)

        def chunk(r, c):
            return plsc.bitcast(noise_v[r, pl.ds(c * _LANES, _LANES)],
                                jnp.int32)

        def pcnt(m):
            return plsc.all_reduce_population_count(m)

        zero16 = jnp.zeros((_LANES,), jnp.int32)
        sent16 = jnp.full((_LANES,), sentinel, jnp.int32)
        rows = range(rows_per_w)
        quarters = range(_SPLITS)

        def full_step(i, carry):
            lo, hi, cb = [list(t) for t in carry]
            mid = [lo[r] + ((hi[r] - lo[r]) >> 1) for r in rows]

            def cbody(c, accs):
                out = []
                for r in rows:
                    a = accs[r]
                    for u in range(4):
                        a = a + pcnt(chunk(r, c * 4 + u) <= mid[r])
                    out.append(a)
                return tuple(out)

            acc = lax.fori_loop(0, n_chunks // 4, cbody,
                                tuple(zero16 for _ in rows))
            for r in rows:
                ge = acc[r] >= K
                lo[r] = jnp.where(ge, lo[r], mid[r] + 1)
                hi[r] = jnp.where(ge, mid[r], hi[r])
                cb[r] = jnp.where(ge, cb[r], acc[r])
            return (tuple(lo), tuple(hi), tuple(cb))

        with jax.named_scope("p1_bisect_full"):
            init = (tuple(zero16 for _ in rows),
                    tuple(jnp.full((_LANES,), sentinel - 1, jnp.int32)
                          for _ in rows),
                    tuple(zero16 for _ in rows))
            lo, hi, cb = [list(t) for t in lax.fori_loop(
                0, _FULL_STEPS, full_step, init)]

        with jax.named_scope("p2_compact"):
            c0 = [cb[r] for r in rows]

            def compact_body(c, offs):
                out = []
                for r in rows:
                    for q in quarters:
                        k = chunk(r, q * q_chunks + c)
                        m = (k >= lo[r]) & (k <= hi[r])
                        o = offs[r * _SPLITS + q]
                        plsc.store_compressed(
                            cand[r][q].at[pl.ds(o, _LANES)], k, mask=m)
                        out.append(o + pcnt(m)[0])
                return tuple(out)

            off_flat = lax.fori_loop(
                0, q_chunks, compact_body,
                tuple(jnp.int32(0) for _ in range(rows_per_w * _SPLITS)))
            off = [[off_flat[r * _SPLITS + q] for q in quarters]
                   for r in rows]
            for r in rows:
                for q in quarters:
                    cand[r][q][pl.ds(off[r][q], _LANES)] = sent16

        with jax.named_scope("p3_select"):
            msum = [functools.reduce(lambda a, b: a + b,
                                     [off[r][q] for q in quarters])
                    for r in rows]
            all_small = (msum[0] <= _LANES) if rows_per_w else None
            for r in rows:
                if r:
                    all_small = all_small & (msum[r] <= _LANES)
            lane = lax.iota(jnp.int32, _LANES)

            def sorted_path(c0s):
                vstars, cbs = [], []
                for r in rows:
                    kw = [cand[r][q][pl.ds(0, _LANES)] for q in quarters]
                    oc = jnp.int32(0)
                    for q in quarters:
                        m = kw[q] < sent16
                        plsc.store_compressed(
                            comb[r].at[pl.ds(oc, _LANES)], kw[q], mask=m)
                        oc = oc + pcnt(m)[0]
                    comb[r][pl.ds(oc, _LANES)] = sent16
                    kall = comb[r][pl.ds(0, _LANES)]
                    ks, _ = plsc.sort_key_val(kall, kall)
                    rprime = K - c0s[r]
                    sel = jnp.where(lane < rprime, ks, 0)
                    vstar = jnp.full((_LANES,), jnp.max(sel), jnp.int32)
                    cbr = c0s[r] + pcnt(kall < vstar)
                    vstars.append(vstar)
                    cbs.append(cbr)
                return tuple(vstars), tuple(cbs)

            def slow_path(c0s):
                nc = off[0][0]
                for r in rows:
                    for q in quarters:
                        if r or q:
                            nc = jnp.maximum(nc, off[r][q])
                nc = (nc + _LANES - 1) // _LANES
                lob = [lo[r] for r in rows]
                hib = [hi[r] for r in rows]

                def step(i, carry):
                    lob, hib, cbs = [list(t) for t in carry]
                    mid = [lob[r] + ((hib[r] - lob[r]) >> 1) for r in rows]

                    def wbody(j, accs):
                        out = []
                        for r in rows:
                            a = accs[r]
                            for q in quarters:
                                kw = cand[r][q][pl.ds(j * _LANES, _LANES)]
                                a = a + pcnt(kw <= mid[r])
                            out.append(a)
                        return tuple(out)

                    accs = lax.fori_loop(0, nc, wbody,
                                         tuple(zero16 for _ in rows))
                    for r in rows:
                        cnt = c0s[r] + accs[r]
                        ge = cnt >= K
                        lob[r] = jnp.where(ge, lob[r], mid[r] + 1)
                        hib[r] = jnp.where(ge, mid[r], hib[r])
                        cbs[r] = jnp.where(ge, cbs[r], cnt)
                    return (tuple(lob), tuple(hib), tuple(cbs))

                lob, _, cbs = lax.fori_loop(0, _TOTAL_STEPS - _FULL_STEPS,
                                            step,
                                            (tuple(lob), tuple(hib),
                                             tuple(c0s)))
                return tuple(lob), cbs

            vstar, cb = lax.cond(all_small, sorted_path, slow_path,
                                 tuple(c0))

        rem = [K - cb[r] for r in rows]

        with jax.named_scope("p4_mask"):
            def mask_body(c, carries):
                out = []
                for r in rows:
                    cy = carries[r]
                    for u in range(2):
                        k = chunk(r, c * 2 + u)
                        eq = k == vstar[r]
                        eqi = eq.astype(jnp.int32)
                        excl = jnp.cumsum(eqi) - eqi + cy
                        vis = (k < vstar[r]) | (eq & (excl < rem[r]))
                        out_v[r, pl.ds((c * 2 + u) * _LANES, _LANES)] = (
                            vis.astype(jnp.float32))
                        cy = cy + pcnt(eq)
                    out.append(cy)
                return tuple(out)

            lax.fori_loop(0, n_chunks // 2, mask_body,
                          tuple(zero16 for _ in rows))

        pltpu.sync_copy(out_v, out_hbm.at[pl.ds(base, rows_per_w)])

    return body


def kernel(x, noise):
    B, N = x.shape[0], x.shape[1]
    num_visible = int(round(N * (1.0 - _MASK_RATIO)))
    num_visible = min(max(1, num_visible), N - 1)
    return _build_mask_kernel(B, N, num_visible)(noise)


# speedup vs baseline: 1.0275x; 1.0275x over previous
import functools

import jax
import jax.numpy as jnp
from jax import lax
from jax.experimental import pallas as pl
from jax.experimental.pallas import tpu as pltpu
from jax.experimental.pallas import tpu_sc as plsc

_MASK_RATIO = 0.75
_LANES = 16
_FULL_STEPS = 13
_TOTAL_STEPS = 30
_SPLITS = 4


@functools.lru_cache(maxsize=None)
def _build_mask_kernel(B, N, K):
    NW = 32
    rows_per_w = B // NW
    n_chunks = N // _LANES
    q_chunks = n_chunks // _SPLITS
    q_elems = N // _SPLITS
    mesh = plsc.VectorSubcoreMesh(core_axis_name="c", subcore_axis_name="s")

    sentinel = 1 << _TOTAL_STEPS
    cand_types = [pltpu.VMEM((q_elems + _LANES,), jnp.int32)
                  for _ in range(rows_per_w * _SPLITS)]
    comb_types = [pltpu.VMEM((2 * _LANES,), jnp.int32)
                  for _ in range(rows_per_w)]

    @functools.partial(
        pl.kernel,
        mesh=mesh,
        out_type=jax.ShapeDtypeStruct((B, N), jnp.float32),
        compiler_params=pltpu.CompilerParams(needs_layout_passes=False),
        scratch_types=[
            pltpu.VMEM((rows_per_w, N), jnp.float32),
            pltpu.VMEM((rows_per_w, N), jnp.float32),
        ] + cand_types + comb_types,
    )
    def body(noise_hbm, out_hbm, noise_v, out_v, *scratch):
        cand = [scratch[r * _SPLITS:(r + 1) * _SPLITS]
                for r in range(rows_per_w)]
        comb = scratch[rows_per_w * _SPLITS:]
        wid = lax.axis_index("s") * 2 + lax.axis_index("c")
        base = wid * rows_per_w
        pltpu.sync_copy(noise_hbm.at[pl.ds(base, rows_per_w)], noise_v)

        def chunk(r, c):
            return plsc.bitcast(noise_v[r, pl.ds(c * _LANES, _LANES)],
                                jnp.int32)

        def pcnt(m):
            return plsc.all_reduce_population_count(m)

        zero16 = jnp.zeros((_LANES,), jnp.int32)
        sent16 = jnp.full((_LANES,), sentinel, jnp.int32)
        rows = range(rows_per_w)
        quarters = range(_SPLITS)

        def full_step(i, carry):
            lo, hi, cb = [list(t) for t in carry]
            mid = [lo[r] + ((hi[r] - lo[r]) >> 1) for r in rows]

            def cbody(c, accs):
                out = []
                for r in rows:
                    a = accs[r]
                    for u in range(4):
                        a = a + pcnt(chunk(r, c * 4 + u) <= mid[r])
                    out.append(a)
                return tuple(out)

            acc = lax.fori_loop(0, n_chunks // 4, cbody,
                                tuple(zero16 for _ in rows))
            for r in rows:
                ge = acc[r] >= K
                lo[r] = jnp.where(ge, lo[r], mid[r] + 1)
                hi[r] = jnp.where(ge, mid[r], hi[r])
                cb[r] = jnp.where(ge, cb[r], acc[r])
            return (tuple(lo), tuple(hi), tuple(cb))

        with jax.named_scope("p1_bisect_full"):
            init = (tuple(zero16 for _ in rows),
                    tuple(jnp.full((_LANES,), sentinel - 1, jnp.int32)
                          for _ in rows),
                    tuple(zero16 for _ in rows))
            lo, hi, cb = [list(t) for t in lax.fori_loop(
                0, _FULL_STEPS, full_step, init)]

        with jax.named_scope("p2_compact"):
            c0 = [cb[r] for r in rows]

            def compact_body(c, offs):
                out = []
                for r in rows:
                    for q in quarters:
                        k = chunk(r, q * q_chunks + c)
                        m = (k >= lo[r]) & (k <= hi[r])
                        o = offs[r * _SPLITS + q]
                        plsc.store_compressed(
                            cand[r][q].at[pl.ds(o, _LANES)], k, mask=m)
                        out.append(o + pcnt(m)[0])
                return tuple(out)

            off_flat = lax.fori_loop(
                0, q_chunks, compact_body,
                tuple(jnp.int32(0) for _ in range(rows_per_w * _SPLITS)))
            off = [[off_flat[r * _SPLITS + q] for q in quarters]
                   for r in rows]
            for r in rows:
                for q in quarters:
                    cand[r][q][pl.ds(off[r][q], _LANES)] = sent16

        with jax.named_scope("p3_select"):
            msum = [functools.reduce(lambda a, b: a + b,
                                     [off[r][q] for q in quarters])
                    for r in rows]
            all_small = (msum[0] <= _LANES) if rows_per_w else None
            for r in rows:
                if r:
                    all_small = all_small & (msum[r] <= _LANES)
            lane = lax.iota(jnp.int32, _LANES)

            def sorted_path(c0s):
                vstars, cbs = [], []
                for r in rows:
                    kw = [cand[r][q][pl.ds(0, _LANES)] for q in quarters]
                    oc = jnp.int32(0)
                    for q in quarters:
                        m = kw[q] < sent16
                        plsc.store_compressed(
                            comb[r].at[pl.ds(oc, _LANES)], kw[q], mask=m)
                        oc = oc + pcnt(m)[0]
                    comb[r][pl.ds(oc, _LANES)] = sent16
                    kall = comb[r][pl.ds(0, _LANES)]
                    ks, _ = plsc.sort_key_val(kall, kall)
                    rprime = K - c0s[r]
                    sel = jnp.where(lane < rprime, ks, 0)
                    vstar = jnp.full((_LANES,), jnp.max(sel), jnp.int32)
                    cbr = c0s[r] + pcnt(kall < vstar)
                    vstars.append(vstar)
                    cbs.append(cbr)
                return tuple(vstars), tuple(cbs)

            def slow_path(c0s):
                nc = off[0][0]
                for r in rows:
                    for q in quarters:
                        if r or q:
                            nc = jnp.maximum(nc, off[r][q])
                nc = (nc + _LANES - 1) // _LANES
                lob = [lo[r] for r in rows]
                hib = [hi[r] for r in rows]

                def step(i, carry):
                    lob, hib, cbs = [list(t) for t in carry]
                    mid = [lob[r] + ((hib[r] - lob[r]) >> 1) for r in rows]

                    def wbody(j, accs):
                        out = []
                        for r in rows:
                            a = accs[r]
                            for q in quarters:
                                kw = cand[r][q][pl.ds(j * _LANES, _LANES)]
                                a = a + pcnt(kw <= mid[r])
                            out.append(a)
                        return tuple(out)

                    accs = lax.fori_loop(0, nc, wbody,
                                         tuple(zero16 for _ in rows))
                    for r in rows:
                        cnt = c0s[r] + accs[r]
                        ge = cnt >= K
                        lob[r] = jnp.where(ge, lob[r], mid[r] + 1)
                        hib[r] = jnp.where(ge, mid[r], hib[r])
                        cbs[r] = jnp.where(ge, cbs[r], cnt)
                    return (tuple(lob), tuple(hib), tuple(cbs))

                lob, _, cbs = lax.fori_loop(0, _TOTAL_STEPS - _FULL_STEPS,
                                            step,
                                            (tuple(lob), tuple(hib),
                                             tuple(c0s)))
                return tuple(lob), cbs

            vstar, cb = lax.cond(all_small, sorted_path, slow_path,
                                 tuple(c0))

        rem = [K - cb[r] for r in rows]

        with jax.named_scope("p4_mask"):
            def mask_body(c, carries):
                out = []
                for r in rows:
                    cy = carries[r]
                    for u in range(2):
                        k = chunk(r, c * 2 + u)
                        eq = k == vstar[r]
                        eqi = eq.astype(jnp.int32)
                        excl = jnp.cumsum(eqi) - eqi + cy
                        vis = (k < vstar[r]) | (eq & (excl < rem[r]))
                        out_v[r, pl.ds((c * 2 + u) * _LANES, _LANES)] = (
                            vis.astype(jnp.float32))
                        cy = cy + pcnt(eq)
                    out.append(cy)
                return tuple(out)

            lax.fori_loop(0, n_chunks // 2, mask_body,
                          tuple(zero16 for _ in rows))

        pltpu.sync_copy(out_v, out_hbm.at[pl.ds(base, rows_per_w)])

    return body


def kernel(x, noise):
    B, N = x.shape[0], x.shape[1]
    num_visible = int(round(N * (1.0 - _MASK_RATIO)))
    num_visible = min(max(1, num_visible), N - 1)
    return _build_mask_kernel(B, N, num_visible)(noise)
